# Initial kernel scaffold; baseline (speedup 1.0000x reference)
import functools

import jax
import jax.numpy as jnp
from jax import lax
from jax.experimental import pallas as pl
from jax.experimental.pallas import tpu as pltpu
from jax.experimental.pallas import tpu_sc as plsc

N = 10000
D = 128
E = 320000
NC = 2
NS = 16
NW = NC * NS
EPW = E // NW
CH = 80
NCH = EPW // CH
RPT = N // NS
DEGW = 16

_mesh = plsc.VectorSubcoreMesh(
    core_axis_name="c", subcore_axis_name="s", num_cores=NC, num_subcores=NS
)


def _deg_body(dst_hbm, ones_hbm, zeros_hbm, out_hbm, dst_v, ones_v, acc, sem):
    c = lax.axis_index("c")
    s = lax.axis_index("s")
    wid = c * NS + s
    base = wid * EPW
    r0 = s * RPT
    pltpu.sync_copy(ones_hbm, ones_v)
    pltpu.sync_copy(zeros_hbm.at[pl.ds(r0, RPT)], acc.at[pl.ds(r0, RPT)])
    plsc.subcore_barrier()

    def body(j, carry):
        pltpu.sync_copy(dst_hbm.at[pl.ds(base + j * CH, CH)], dst_v)
        pltpu.sync_copy(ones_v, acc.at[dst_v], add=True)
        return carry

    lax.fori_loop(0, NCH, body, 0)
    plsc.subcore_barrier()
    pltpu.sync_copy(acc.at[pl.ds(r0, RPT)],
                    out_hbm.at[pl.ds(c * N + r0, RPT)])


_deg_kernel = functools.partial(
    pl.kernel,
    out_type=jax.ShapeDtypeStruct((NC * N, DEGW), jnp.float32),
    mesh=_mesh,
    scratch_types=[
        pltpu.VMEM((CH,), jnp.int32),
        pltpu.VMEM((CH, DEGW), jnp.float32),
        pltpu.VMEM_SHARED((N, DEGW), jnp.float32),
        pltpu.SemaphoreType.DMA,
    ],
)(_deg_body)


def _agg_body(hp_hbm, src_hbm, dst_hbm, zeros_hbm, out_hbm,
              src_v, dst_v, rows_v, acc, sem):
    c = lax.axis_index("c")
    s = lax.axis_index("s")
    wid = c * NS + s
    base = wid * EPW
    r0 = s * RPT
    pltpu.sync_copy(zeros_hbm.at[pl.ds(r0, RPT)], acc.at[pl.ds(r0, RPT)])
    plsc.subcore_barrier()

    def body(j, carry):
        e0 = base + j * CH
        pltpu.sync_copy(src_hbm.at[pl.ds(e0, CH)], src_v)
        pltpu.sync_copy(dst_hbm.at[pl.ds(e0, CH)], dst_v)
        pltpu.async_copy(hp_hbm.at[src_v], rows_v, sem).wait()
        pltpu.sync_copy(rows_v, acc.at[dst_v], add=True)
        return carry

    lax.fori_loop(0, NCH, body, 0)
    plsc.subcore_barrier()
    pltpu.sync_copy(acc.at[pl.ds(r0, RPT)],
                    out_hbm.at[pl.ds(c * N + r0, RPT)])


_agg_kernel = functools.partial(
    pl.kernel,
    out_type=jax.ShapeDtypeStruct((NC * N, D), jnp.float32),
    mesh=_mesh,
    scratch_types=[
        pltpu.VMEM((CH,), jnp.int32),
        pltpu.VMEM((CH,), jnp.int32),
        pltpu.VMEM((CH, D), jnp.float32),
        pltpu.VMEM_SHARED((N, D), jnp.float32),
        pltpu.SemaphoreType.DMA,
    ],
)(_agg_body)


BLK = 1000


def _dinv_block(degs0, degs1):
    return lax.rsqrt(degs0[:, 0:1] + degs1[:, 0:1] + 1.0)


def _layer1_body(x_ref, w_ref, dg0_ref, dg1_ref, o_ref):
    dinv = _dinv_block(dg0_ref[...], dg1_ref[...])
    h = jnp.dot(x_ref[...], w_ref[...], preferred_element_type=jnp.float32)
    o_ref[...] = h * dinv


def _layer2_body(a0_ref, a1_ref, hp_ref, dg0_ref, dg1_ref, b_ref, w_ref,
                 o_ref):
    dinv = _dinv_block(dg0_ref[...], dg1_ref[...])
    z = dinv * (a0_ref[...] + a1_ref[...] + hp_ref[...]) + b_ref[...]
    z = jnp.maximum(z, 0.0)
    o_ref[...] = jnp.dot(z, w_ref[...],
                         preferred_element_type=jnp.float32) * dinv


def _head_body(a_ref, hp_ref, dg_ref, b_ref, l1w_ref, l1b_ref, l2w_ref,
               l2b_ref, l3w_ref, l3b_ref, o_ref):
    dinv = lax.rsqrt(dg_ref[0:N, 0:1] + dg_ref[N:2 * N, 0:1] + 1.0)
    z = dinv * (a_ref[0:N] + a_ref[N:2 * N] + hp_ref[...]) + b_ref[...]
    z = jnp.maximum(z, 0.0)
    g = jnp.mean(z, axis=0, keepdims=True)
    g = jnp.maximum(
        jnp.dot(g, l1w_ref[...], preferred_element_type=jnp.float32)
        + l1b_ref[...], 0.0)
    g = jnp.maximum(
        jnp.dot(g, l2w_ref[...], preferred_element_type=jnp.float32)
        + l2b_ref[...], 0.0)
    o_ref[...] = jnp.dot(g, l3w_ref[...],
                         preferred_element_type=jnp.float32) + l3b_ref[...]


def _row_spec(w):
    return pl.BlockSpec((BLK, w), lambda i: (i, 0))


def _row_spec_hi(w):
    return pl.BlockSpec((BLK, w), lambda i: (i + N // BLK, 0))


def _full_spec(shape):
    return pl.BlockSpec(shape, lambda i: tuple(0 for _ in shape))


_layer1 = pl.pallas_call(
    _layer1_body,
    grid=(N // BLK,),
    in_specs=[
        _row_spec(D),
        _full_spec((D, D)),
        _row_spec(DEGW),
        _row_spec_hi(DEGW),
    ],
    out_specs=_row_spec(D),
    out_shape=jax.ShapeDtypeStruct((N, D), jnp.float32),
)

_layer2 = pl.pallas_call(
    _layer2_body,
    grid=(N // BLK,),
    in_specs=[
        _row_spec(D),
        _row_spec_hi(D),
        _row_spec(D),
        _row_spec(DEGW),
        _row_spec_hi(DEGW),
        _full_spec((1, D)),
        _full_spec((D, D)),
    ],
    out_specs=_row_spec(D),
    out_shape=jax.ShapeDtypeStruct((N, D), jnp.float32),
)

_head = pl.pallas_call(
    _head_body,
    out_shape=jax.ShapeDtypeStruct((1, 1), jnp.float32),
)


def kernel(x, edge_index, W0, b0, W1, b1, L1w, L1b, L2w, L2b, L3w, L3b):
    src = edge_index[0].astype(jnp.int32)
    dst = edge_index[1].astype(jnp.int32)
    ones16 = jnp.ones((CH, DEGW), jnp.float32)
    zeros16 = jnp.zeros((N, DEGW), jnp.float32)
    zeros128 = jnp.zeros((N, D), jnp.float32)

    degs = _deg_kernel(dst, ones16, zeros16)
    hp1 = _layer1(x, W0, degs, degs)
    p = _agg_kernel(hp1, src, dst, zeros128)
    hp2 = _layer2(p, p, hp1, degs, degs, b0.reshape(1, D), W1)
    q = _agg_kernel(hp2, src, dst, zeros128)
    out = _head(q, hp2, degs, b1.reshape(1, D),
                L1w, L1b.reshape(1, D), L2w, L2b.reshape(1, D),
                L3w, L3b.reshape(1, 1))
    return out


# speedup vs baseline: 12.2500x; 12.2500x over previous
import functools

import jax
import jax.numpy as jnp
from jax import lax
from jax.experimental import pallas as pl
from jax.experimental.pallas import tpu as pltpu
from jax.experimental.pallas import tpu_sc as plsc

N = 10000
D = 128
E = 320000
NC = 2
NS = 16
NW = NC * NS
EPW = E // NW
CH = 80
NCH = EPW // CH
RPT = 624
TAIL = N - NS * RPT
TAIL0 = NS * RPT
DEGW = 128


def _zero_fill(zeros_hbm, acc, s):
    r0 = s * RPT
    pltpu.sync_copy(zeros_hbm.at[pl.ds(r0, RPT)], acc.at[pl.ds(r0, RPT)])

    @pl.when(s == 0)
    def _():
        pltpu.sync_copy(zeros_hbm.at[pl.ds(TAIL0, TAIL)],
                        acc.at[pl.ds(TAIL0, TAIL)])


def _write_out(acc, out_hbm, c, s):
    r0 = s * RPT
    pltpu.sync_copy(acc.at[pl.ds(r0, RPT)],
                    out_hbm.at[pl.ds(c * N + r0, RPT)])

    @pl.when(s == 0)
    def _():
        pltpu.sync_copy(acc.at[pl.ds(TAIL0, TAIL)],
                        out_hbm.at[pl.ds(c * N + TAIL0, TAIL)])


def _deg_body(dst_hbm, ones_hbm, zeros_hbm, out_hbm, dst_v, ones_v, acc, sem):
    c = lax.axis_index("c")
    s = lax.axis_index("s")
    wid = c * NS + s
    base = wid * EPW
    pltpu.sync_copy(ones_hbm, ones_v)
    _zero_fill(zeros_hbm, acc, s)
    plsc.subcore_barrier()

    def body(j, carry):
        pltpu.sync_copy(dst_hbm.at[pl.ds(base + j * CH, CH)], dst_v)
        pltpu.sync_copy(ones_v, acc.at[dst_v], add=True)
        return carry

    lax.fori_loop(0, NCH, body, 0)
    plsc.subcore_barrier()
    _write_out(acc, out_hbm, c, s)


@functools.cache
def _sc_kernels():
    mesh = plsc.VectorSubcoreMesh(
        core_axis_name="c", subcore_axis_name="s",
        num_cores=NC, num_subcores=NS,
    )
    deg = pl.kernel(
        _deg_body,
        out_type=jax.ShapeDtypeStruct((NC * N, DEGW), jnp.float32),
        mesh=mesh,
        scratch_types=[
            pltpu.VMEM((CH,), jnp.int32),
            pltpu.VMEM((CH, DEGW), jnp.float32),
            pltpu.VMEM_SHARED((N, DEGW), jnp.float32),
            pltpu.SemaphoreType.DMA,
        ],
    )
    agg = pl.kernel(
        _agg_body,
        out_type=jax.ShapeDtypeStruct((NC * N, D), jnp.float32),
        mesh=mesh,
        scratch_types=[
            pltpu.VMEM((CH,), jnp.int32),
            pltpu.VMEM((CH,), jnp.int32),
            pltpu.VMEM((CH, D), jnp.float32),
            pltpu.VMEM_SHARED((N, D), jnp.float32),
            pltpu.SemaphoreType.DMA,
        ],
    )
    return deg, agg


def _agg_body(hp_hbm, src_hbm, dst_hbm, zeros_hbm, out_hbm,
              src_v, dst_v, rows_v, acc, sem):
    c = lax.axis_index("c")
    s = lax.axis_index("s")
    wid = c * NS + s
    base = wid * EPW
    _zero_fill(zeros_hbm, acc, s)
    plsc.subcore_barrier()

    def body(j, carry):
        e0 = base + j * CH
        pltpu.sync_copy(src_hbm.at[pl.ds(e0, CH)], src_v)
        pltpu.sync_copy(dst_hbm.at[pl.ds(e0, CH)], dst_v)
        pltpu.async_copy(hp_hbm.at[src_v], rows_v, sem).wait()
        pltpu.sync_copy(rows_v, acc.at[dst_v], add=True)
        return carry

    lax.fori_loop(0, NCH, body, 0)
    plsc.subcore_barrier()
    _write_out(acc, out_hbm, c, s)


BLK = 1000


def _dinv_block(degs0, degs1):
    return lax.rsqrt(degs0[:, 0:1] + degs1[:, 0:1] + 1.0)


def _layer1_body(x_ref, w_ref, dg0_ref, dg1_ref, o_ref):
    dinv = _dinv_block(dg0_ref[...], dg1_ref[...])
    h = jnp.dot(x_ref[...], w_ref[...], preferred_element_type=jnp.float32)
    o_ref[...] = h * dinv


def _layer2_body(a0_ref, a1_ref, hp_ref, dg0_ref, dg1_ref, b_ref, w_ref,
                 o_ref):
    dinv = _dinv_block(dg0_ref[...], dg1_ref[...])
    z = dinv * (a0_ref[...] + a1_ref[...] + hp_ref[...]) + b_ref[...]
    z = jnp.maximum(z, 0.0)
    o_ref[...] = jnp.dot(z, w_ref[...],
                         preferred_element_type=jnp.float32) * dinv


def _head_body(a_ref, hp_ref, dg_ref, b_ref, l1w_ref, l1b_ref, l2w_ref,
               l2b_ref, l3w_ref, l3b_ref, o_ref):
    dinv = lax.rsqrt(dg_ref[0:N, 0:1] + dg_ref[N:2 * N, 0:1] + 1.0)
    z = dinv * (a_ref[0:N] + a_ref[N:2 * N] + hp_ref[...]) + b_ref[...]
    z = jnp.maximum(z, 0.0)
    g = jnp.mean(z, axis=0, keepdims=True)
    g = jnp.maximum(
        jnp.dot(g, l1w_ref[...], preferred_element_type=jnp.float32)
        + l1b_ref[...], 0.0)
    g = jnp.maximum(
        jnp.dot(g, l2w_ref[...], preferred_element_type=jnp.float32)
        + l2b_ref[...], 0.0)
    o_ref[...] = jnp.dot(g, l3w_ref[...],
                         preferred_element_type=jnp.float32) + l3b_ref[...]


def _row_spec(w):
    return pl.BlockSpec((BLK, w), lambda i: (i, 0))


def _row_spec_hi(w):
    return pl.BlockSpec((BLK, w), lambda i: (i + N // BLK, 0))


def _full_spec(shape):
    return pl.BlockSpec(shape, lambda i: tuple(0 for _ in shape))


_layer1 = pl.pallas_call(
    _layer1_body,
    grid=(N // BLK,),
    in_specs=[
        _row_spec(D),
        _full_spec((D, D)),
        _row_spec(DEGW),
        _row_spec_hi(DEGW),
    ],
    out_specs=_row_spec(D),
    out_shape=jax.ShapeDtypeStruct((N, D), jnp.float32),
)

_layer2 = pl.pallas_call(
    _layer2_body,
    grid=(N // BLK,),
    in_specs=[
        _row_spec(D),
        _row_spec_hi(D),
        _row_spec(D),
        _row_spec(DEGW),
        _row_spec_hi(DEGW),
        _full_spec((1, D)),
        _full_spec((D, D)),
    ],
    out_specs=_row_spec(D),
    out_shape=jax.ShapeDtypeStruct((N, D), jnp.float32),
)

_head = pl.pallas_call(
    _head_body,
    out_shape=jax.ShapeDtypeStruct((1, 1), jnp.float32),
)


def kernel(x, edge_index, W0, b0, W1, b1, L1w, L1b, L2w, L2b, L3w, L3b):
    src = edge_index[0].astype(jnp.int32)
    dst = edge_index[1].astype(jnp.int32)
    ones16 = jnp.ones((CH, DEGW), jnp.float32)
    zeros16 = jnp.zeros((N, DEGW), jnp.float32)
    zeros128 = jnp.zeros((N, D), jnp.float32)

    _deg_kernel, _agg_kernel = _sc_kernels()
    degs = _deg_kernel(dst, ones16, zeros16)
    hp1 = _layer1(x, W0, degs, degs)
    p = _agg_kernel(hp1, src, dst, zeros128)
    hp2 = _layer2(p, p, hp1, degs, degs, b0.reshape(1, D), W1)
    q = _agg_kernel(hp2, src, dst, zeros128)
    out = _head(q, hp2, degs, b1.reshape(1, D),
                L1w, L1b.reshape(1, D), L2w, L2b.reshape(1, D),
                L3w, L3b.reshape(1, 1))
    return out


# speedup vs baseline: 19.9723x; 1.6304x over previous
import functools

import jax
import jax.numpy as jnp
from jax import lax
from jax.experimental import pallas as pl
from jax.experimental.pallas import tpu as pltpu
from jax.experimental.pallas import tpu_sc as plsc

N = 10000
D = 128
E = 320000
NC = 2
NS = 16
NW = NC * NS
EPW = E // NW
CH = 80
NCH = EPW // CH
RPT = 624
TAIL = N - NS * RPT
TAIL0 = NS * RPT
DEGW = 128


def _zero_fill(zeros_hbm, acc, s):
    r0 = s * RPT
    pltpu.sync_copy(zeros_hbm.at[pl.ds(r0, RPT)], acc.at[pl.ds(r0, RPT)])

    @pl.when(s == 0)
    def _():
        pltpu.sync_copy(zeros_hbm.at[pl.ds(TAIL0, TAIL)],
                        acc.at[pl.ds(TAIL0, TAIL)])


def _write_out(acc, out_hbm, c, s):
    r0 = s * RPT
    pltpu.sync_copy(acc.at[pl.ds(r0, RPT)],
                    out_hbm.at[pl.ds(c * N + r0, RPT)])

    @pl.when(s == 0)
    def _():
        pltpu.sync_copy(acc.at[pl.ds(TAIL0, TAIL)],
                        out_hbm.at[pl.ds(c * N + TAIL0, TAIL)])


NBUF = 4
NG = NCH // NBUF


def _deg_body(dst_hbm, ones_hbm, zeros_hbm, out_hbm,
              d0, d1, d2, d3, ones_v, acc,
              s0, s1, s2, s3):
    c = lax.axis_index("c")
    s = lax.axis_index("s")
    base = (c * NS + s) * EPW
    dsts = (d0, d1, d2, d3)
    sems = (s0, s1, s2, s3)
    pltpu.sync_copy(ones_hbm, ones_v)
    _zero_fill(zeros_hbm, acc, s)
    plsc.subcore_barrier()

    def load_idx(j, b):
        pltpu.sync_copy(dst_hbm.at[pl.ds(base + j * CH, CH)], dsts[b])

    def scatter(b):
        pltpu.async_copy(ones_v, acc.at[dsts[b]], sems[b], add=True)

    def wait_scatter(b):
        pltpu.make_async_copy(ones_v, acc.at[dsts[b]], sems[b]).wait()

    for b in range(NBUF):
        load_idx(b, b)

    def body(t, carry):
        for b in range(NBUF):
            scatter(b)
        for b in range(NBUF):
            jn = (t + 1) * NBUF + b

            @pl.when(jn < NCH)
            def _():
                wait_scatter(b)
                load_idx(jn, b)
        return carry

    lax.fori_loop(0, NG, body, 0)
    scatter(0)
    for b in range(NBUF):
        wait_scatter(b)
    plsc.subcore_barrier()
    _write_out(acc, out_hbm, c, s)


@functools.cache
def _sc_kernels():
    mesh = plsc.VectorSubcoreMesh(
        core_axis_name="c", subcore_axis_name="s",
        num_cores=NC, num_subcores=NS,
    )
    deg = pl.kernel(
        _deg_body,
        out_type=jax.ShapeDtypeStruct((NC * N, DEGW), jnp.float32),
        mesh=mesh,
        scratch_types=(
            [pltpu.VMEM((CH,), jnp.int32)] * NBUF
            + [pltpu.VMEM((CH, DEGW), jnp.float32),
               pltpu.VMEM_SHARED((N, DEGW), jnp.float32)]
            + [pltpu.SemaphoreType.DMA] * NBUF
        ),
    )
    agg = pl.kernel(
        _agg_body,
        out_type=jax.ShapeDtypeStruct((NC * N, D), jnp.float32),
        mesh=mesh,
        scratch_types=(
            [pltpu.VMEM((CH,), jnp.int32)] * (2 * NBUF)
            + [pltpu.VMEM((CH, D), jnp.float32)] * NBUF
            + [pltpu.VMEM_SHARED((N, D), jnp.float32)]
            + [pltpu.SemaphoreType.DMA] * (2 * NBUF)
        ),
    )
    return deg, agg


def _agg_body(hp_hbm, src_hbm, dst_hbm, zeros_hbm, out_hbm,
              sv0, sv1, sv2, sv3, dv0, dv1, dv2, dv3,
              r0, r1, r2, r3, acc,
              g0, g1, g2, g3, t0, t1, t2, t3):
    c = lax.axis_index("c")
    s = lax.axis_index("s")
    base = (c * NS + s) * EPW
    srcs = (sv0, sv1, sv2, sv3)
    dsts = (dv0, dv1, dv2, dv3)
    rows = (r0, r1, r2, r3)
    gsem = (g0, g1, g2, g3)
    ssem = (t0, t1, t2, t3)
    _zero_fill(zeros_hbm, acc, s)
    plsc.subcore_barrier()

    def load_and_gather(j, b):
        e0 = base + j * CH
        pltpu.sync_copy(src_hbm.at[pl.ds(e0, CH)], srcs[b])
        pltpu.sync_copy(dst_hbm.at[pl.ds(e0, CH)], dsts[b])
        pltpu.async_copy(hp_hbm.at[srcs[b]], rows[b], gsem[b])

    def wait_gather(b):
        pltpu.make_async_copy(hp_hbm.at[srcs[b]], rows[b], gsem[b]).wait()

    def scatter(b):
        pltpu.async_copy(rows[b], acc.at[dsts[b]], ssem[b], add=True)

    def wait_scatter(b):
        pltpu.make_async_copy(rows[b], acc.at[dsts[b]], ssem[b]).wait()

    for b in range(NBUF):
        load_and_gather(b, b)

    def body(t, carry):
        for b in range(NBUF):
            wait_gather(b)
            scatter(b)
        for b in range(NBUF):
            jn = (t + 1) * NBUF + b

            @pl.when(jn < NCH)
            def _():
                wait_scatter(b)
                load_and_gather(jn, b)
        return carry

    lax.fori_loop(0, NG, body, 0)
    wait_gather(0)
    scatter(0)
    for b in range(NBUF):
        wait_scatter(b)
    plsc.subcore_barrier()
    _write_out(acc, out_hbm, c, s)


BLK = 1000


def _dinv_block(degs0, degs1):
    return lax.rsqrt(degs0[:, 0:1] + degs1[:, 0:1] + 1.0)


def _layer1_body(x_ref, w_ref, dg0_ref, dg1_ref, o_ref):
    dinv = _dinv_block(dg0_ref[...], dg1_ref[...])
    h = jnp.dot(x_ref[...], w_ref[...], preferred_element_type=jnp.float32)
    o_ref[...] = h * dinv


def _layer2_body(a0_ref, a1_ref, hp_ref, dg0_ref, dg1_ref, b_ref, w_ref,
                 o_ref):
    dinv = _dinv_block(dg0_ref[...], dg1_ref[...])
    z = dinv * (a0_ref[...] + a1_ref[...] + hp_ref[...]) + b_ref[...]
    z = jnp.maximum(z, 0.0)
    o_ref[...] = jnp.dot(z, w_ref[...],
                         preferred_element_type=jnp.float32) * dinv


def _head_body(a_ref, hp_ref, dg_ref, b_ref, l1w_ref, l1b_ref, l2w_ref,
               l2b_ref, l3w_ref, l3b_ref, o_ref):
    dinv = lax.rsqrt(dg_ref[0:N, 0:1] + dg_ref[N:2 * N, 0:1] + 1.0)
    z = dinv * (a_ref[0:N] + a_ref[N:2 * N] + hp_ref[...]) + b_ref[...]
    z = jnp.maximum(z, 0.0)
    g = jnp.mean(z, axis=0, keepdims=True)
    g = jnp.maximum(
        jnp.dot(g, l1w_ref[...], preferred_element_type=jnp.float32)
        + l1b_ref[...], 0.0)
    g = jnp.maximum(
        jnp.dot(g, l2w_ref[...], preferred_element_type=jnp.float32)
        + l2b_ref[...], 0.0)
    o_ref[...] = jnp.dot(g, l3w_ref[...],
                         preferred_element_type=jnp.float32) + l3b_ref[...]


def _row_spec(w):
    return pl.BlockSpec((BLK, w), lambda i: (i, 0))


def _row_spec_hi(w):
    return pl.BlockSpec((BLK, w), lambda i: (i + N // BLK, 0))


def _full_spec(shape):
    return pl.BlockSpec(shape, lambda i: tuple(0 for _ in shape))


_layer1 = pl.pallas_call(
    _layer1_body,
    grid=(N // BLK,),
    in_specs=[
        _row_spec(D),
        _full_spec((D, D)),
        _row_spec(DEGW),
        _row_spec_hi(DEGW),
    ],
    out_specs=_row_spec(D),
    out_shape=jax.ShapeDtypeStruct((N, D), jnp.float32),
)

_layer2 = pl.pallas_call(
    _layer2_body,
    grid=(N // BLK,),
    in_specs=[
        _row_spec(D),
        _row_spec_hi(D),
        _row_spec(D),
        _row_spec(DEGW),
        _row_spec_hi(DEGW),
        _full_spec((1, D)),
        _full_spec((D, D)),
    ],
    out_specs=_row_spec(D),
    out_shape=jax.ShapeDtypeStruct((N, D), jnp.float32),
)

_head = pl.pallas_call(
    _head_body,
    out_shape=jax.ShapeDtypeStruct((1, 1), jnp.float32),
)


def kernel(x, edge_index, W0, b0, W1, b1, L1w, L1b, L2w, L2b, L3w, L3b):
    src = edge_index[0].astype(jnp.int32)
    dst = edge_index[1].astype(jnp.int32)
    ones16 = jnp.ones((CH, DEGW), jnp.float32)
    zeros16 = jnp.zeros((N, DEGW), jnp.float32)
    zeros128 = jnp.zeros((N, D), jnp.float32)

    _deg_kernel, _agg_kernel = _sc_kernels()
    degs = _deg_kernel(dst, ones16, zeros16)
    hp1 = _layer1(x, W0, degs, degs)
    p = _agg_kernel(hp1, src, dst, zeros128)
    hp2 = _layer2(p, p, hp1, degs, degs, b0.reshape(1, D), W1)
    q = _agg_kernel(hp2, src, dst, zeros128)
    out = _head(q, hp2, degs, b1.reshape(1, D),
                L1w, L1b.reshape(1, D), L2w, L2b.reshape(1, D),
                L3w, L3b.reshape(1, 1))
    return out
